# Initial kernel scaffold; baseline (speedup 1.0000x reference)
import jax
import jax.numpy as jnp
from jax.experimental import pallas as pl


def kernel(x, edge_index, W_pool, W_embed):
    raise NotImplementedError("write your pallas kernel here")


# speedup vs baseline: 3.7740x; 3.7740x over previous
import functools

import jax
import jax.numpy as jnp
from jax import lax
from jax.experimental import pallas as pl
from jax.experimental.pallas import tpu as pltpu
from jax.experimental.pallas import tpu_sc as plsc

N = 10000
D = 128
C = 500
CPAD = 512
W = 64
XBLK = 2
SBLK = 8
NPAD = 10240
ROWS_PER_TILE = NPAD // 16
BATCH = 128
ROW_TILE = 1000

_mesh = plsc.VectorSubcoreMesh(core_axis_name="c", subcore_axis_name="s")


def _zero_zbuf(zbuf):
    zero16 = jnp.zeros((16,), jnp.float32)
    per_row = W // 16

    def body(i, carry):
        r = i // per_row
        j = (i % per_row) * 16
        zbuf[r, pl.ds(j, 16)] = zero16
        return carry

    lax.fori_loop(0, BATCH * per_row, body, 0)


def _make_sc_spmm(nbt, per_sc, nb):
    @functools.partial(
        pl.kernel,
        out_type=jax.ShapeDtypeStruct((nbt * NPAD, W), jnp.float32),
        mesh=_mesh,
        compiler_params=pltpu.CompilerParams(use_tc_tiling_on_sc=False),
        scratch_types=[
            pltpu.VMEM((nb, BATCH), jnp.int32),
            pltpu.VMEM((nb, BATCH), jnp.int32),
            pltpu.VMEM((BATCH, W), jnp.float32),
            pltpu.VMEM((BATCH, W), jnp.float32),
            pltpu.VMEM_SHARED((NPAD, W), jnp.float32),
        ],
    )
    def k(tab_hbm, src_hbm, dst_hbm, out_hbm, src_v, dst_v, rows, zbuf, acc):
        cid = lax.axis_index("c")
        sid = lax.axis_index("s")
        pltpu.sync_copy(dst_hbm.at[sid], dst_v)
        _zero_zbuf(zbuf)

        def do_block(blk):
            for kk in range(ROWS_PER_TILE // BATCH):
                pltpu.sync_copy(
                    zbuf, acc.at[pl.ds(sid * ROWS_PER_TILE + kk * BATCH, BATCH)])
            pltpu.sync_copy(src_hbm.at[blk * 16 + sid], src_v)
            plsc.subcore_barrier()

            def body(b, carry):
                pltpu.sync_copy(tab_hbm.at[src_v.at[b]], rows)
                pltpu.sync_copy(rows, acc.at[dst_v.at[b]], add=True)
                return carry

            lax.fori_loop(0, nb, body, 0)
            plsc.subcore_barrier()
            pltpu.sync_copy(
                acc.at[pl.ds(sid * ROWS_PER_TILE, ROWS_PER_TILE)],
                out_hbm.at[pl.ds(blk * NPAD + sid * ROWS_PER_TILE, ROWS_PER_TILE)],
            )
            plsc.subcore_barrier()

        for j in range(per_sc):
            do_block(cid * per_sc + j)

    return k


def _tc_pool_embed(p_ref, wp_ref, we_ref, s_ref, cx_ref):
    i = pl.program_id(0)
    ax = jnp.concatenate([p_ref[0], p_ref[1]], axis=1)
    h = lax.dot_general(ax, wp_ref[...], (((1,), (0,)), ((), ())),
                        preferred_element_type=jnp.float32)
    h = jnp.maximum(h, 0.0)
    col = lax.broadcasted_iota(jnp.int32, (ROW_TILE, CPAD), 1)
    hm = jnp.where(col < C, h, -1e30)
    m = jnp.max(hm, axis=1, keepdims=True)
    e = jnp.exp(hm - m)
    s = e / jnp.sum(e, axis=1, keepdims=True)
    z = jnp.maximum(
        lax.dot_general(ax, we_ref[...], (((1,), (0,)), ((), ())),
                        preferred_element_type=jnp.float32), 0.0)
    for b in range(SBLK):
        s_ref[b] = s[:, b * W:(b + 1) * W]
    contrib = lax.dot_general(s, z, (((0,), (0,)), ((), ())),
                              preferred_element_type=jnp.float32)

    @pl.when(i == 0)
    def _():
        cx_ref[...] = contrib

    @pl.when(i > 0)
    def _():
        cx_ref[...] += contrib


def _tc_coarse_a(s_ref, as_ref, ca_ref):
    i = pl.program_id(0)
    s = jnp.concatenate([s_ref[b] for b in range(SBLK)], axis=1)
    a = jnp.concatenate([as_ref[b] for b in range(SBLK)], axis=1)
    contrib = lax.dot_general(s, a, (((0,), (0,)), ((), ())),
                              preferred_element_type=jnp.float32)

    @pl.when(i == 0)
    def _():
        ca_ref[...] = contrib

    @pl.when(i > 0)
    def _():
        ca_ref[...] += contrib


def kernel(x, edge_index, W_pool, W_embed):
    E = edge_index.shape[1]
    epad = -E % (16 * BATCH)
    src = jnp.concatenate([edge_index[0], jnp.zeros((epad,), jnp.int32)])
    dst = jnp.concatenate([edge_index[1], jnp.full((epad,), N, jnp.int32)])
    nb = (E + epad) // (16 * BATCH)

    dst16 = dst.reshape(16, nb, BATCH)
    src_x = (src[None, :] + (jnp.arange(XBLK, dtype=jnp.int32) * N)[:, None]
             ).reshape(XBLK * 16, nb, BATCH)
    src_s = (src[None, :] + (jnp.arange(SBLK, dtype=jnp.int32) * NPAD)[:, None]
             ).reshape(SBLK * 16, nb, BATCH)

    xb = jnp.stack([x[:, :W], x[:, W:]]).reshape(XBLK * N, W)
    axb = _make_sc_spmm(XBLK, 1, nb)(xb, src_x, dst16).reshape(XBLK, NPAD, W)

    wp_pad = jnp.zeros((D, CPAD), jnp.float32).at[:, :C].set(W_pool)

    grid = N // ROW_TILE
    s_blk, cx = pl.pallas_call(
        _tc_pool_embed,
        grid=(grid,),
        in_specs=[
            pl.BlockSpec((XBLK, ROW_TILE, W), lambda i: (0, i, 0)),
            pl.BlockSpec((D, CPAD), lambda i: (0, 0)),
            pl.BlockSpec((D, D), lambda i: (0, 0)),
        ],
        out_specs=[
            pl.BlockSpec((SBLK, ROW_TILE, W), lambda i: (0, i, 0)),
            pl.BlockSpec((CPAD, D), lambda i: (0, 0)),
        ],
        out_shape=[
            jax.ShapeDtypeStruct((SBLK, NPAD, W), jnp.float32),
            jax.ShapeDtypeStruct((CPAD, D), jnp.float32),
        ],
    )(axb, wp_pad, W_embed)

    as_flat = _make_sc_spmm(SBLK, SBLK // 2, nb)(
        s_blk.reshape(SBLK * NPAD, W), src_s, dst16)

    ca = pl.pallas_call(
        _tc_coarse_a,
        grid=(grid,),
        in_specs=[
            pl.BlockSpec((SBLK, ROW_TILE, W), lambda i: (0, i, 0)),
            pl.BlockSpec((SBLK, ROW_TILE, W), lambda i: (0, i, 0)),
        ],
        out_specs=pl.BlockSpec((CPAD, CPAD), lambda i: (0, 0)),
        out_shape=jax.ShapeDtypeStruct((CPAD, CPAD), jnp.float32),
    )(s_blk, as_flat.reshape(SBLK, NPAD, W))

    return (ca[:C, :C], cx[:C, :])
